# baseline (speedup 1.0000x reference)
import functools
import math

import jax
import jax.numpy as jnp
from jax import lax
from jax.experimental import pallas as pl
from jax.experimental.pallas import tpu as pltpu
from jax.experimental.pallas import tpu_sc as plsc

_B, _D, _H, _E, _K = 2048, 1024, 2048, 8, 2
_M = 256
_TMAX = 24
_R = _TMAX * _M
_A = _B * _K
_NW = 32
_EPAD = 128


def _gate_body(x_ref, wg_ref, bg_ref, e1_ref, e2_ref, w1_ref, w2_ref,
               aux_ref):
    s = jnp.dot(x_ref[...], wg_ref[...],
                preferred_element_type=jnp.float32) + bg_ref[...]
    col = lax.broadcasted_iota(jnp.int32, s.shape, 1)
    big = jnp.int32(2 ** 30)
    m1 = jnp.max(s, axis=1, keepdims=True)
    e1 = jnp.min(jnp.where(s >= m1, col, big), axis=1, keepdims=True)
    s2 = jnp.where(col == e1, -jnp.inf, s)
    m2 = jnp.max(s2, axis=1, keepdims=True)
    e2 = jnp.min(jnp.where(s2 >= m2, col, big), axis=1, keepdims=True)
    p = jnp.exp(m2 - m1)
    e1_ref[...] = e1
    e2_ref[...] = e2
    w1_ref[...] = 1.0 / (1.0 + p)
    w2_ref[...] = p / (1.0 + p)
    on = (jnp.where(col == e1, 1.0, 0.0) + jnp.where(col == e2, 1.0, 0.0))
    frac = jnp.sum(on, axis=0, keepdims=True) * (1.0 / (_B * _K))
    d = jnp.where(col[0:1, :] < _E, (frac - 1.0 / _E) ** 2, 0.0)
    aux_ref[0, 0] = jnp.sum(d) * (1.0 / _E)


def _gate(x, wg_pad, bg_pad):
    return pl.pallas_call(
        _gate_body,
        out_shape=(
            jax.ShapeDtypeStruct((_B, 1), jnp.int32),
            jax.ShapeDtypeStruct((_B, 1), jnp.int32),
            jax.ShapeDtypeStruct((_B, 1), jnp.float32),
            jax.ShapeDtypeStruct((_B, 1), jnp.float32),
            jax.ShapeDtypeStruct((1, 1), jnp.float32),
        ),
        out_specs=(
            pl.BlockSpec((_B, 1), lambda: (0, 0)),
            pl.BlockSpec((_B, 1), lambda: (0, 0)),
            pl.BlockSpec((_B, 1), lambda: (0, 0)),
            pl.BlockSpec((_B, 1), lambda: (0, 0)),
            pl.BlockSpec(memory_space=pltpu.SMEM),
        ),
    )(x, wg_pad, bg_pad)


def _mlp_body(meta_ref, xs_ref, w1_ref, b1_ref, w2_ref, b2_ref, out_ref):
    t = pl.program_id(0)

    @pl.when(meta_ref[1, t] == 1)
    def _():
        xg = xs_ref[...].astype(jnp.bfloat16)
        h = jnp.dot(xg, w1_ref[0], preferred_element_type=jnp.float32)
        h = h + b1_ref[0]
        h = 0.5 * h * (1.0 + lax.erf(h * (1.0 / math.sqrt(2.0))))
        y = jnp.dot(h.astype(jnp.bfloat16), w2_ref[0],
                    preferred_element_type=jnp.float32)
        out_ref[...] = y + b2_ref[0]


def _grouped_mlp(meta, xs, w1, b1, w2, b2):
    grid_spec = pltpu.PrefetchScalarGridSpec(
        num_scalar_prefetch=1,
        grid=(_TMAX,),
        in_specs=[
            pl.BlockSpec((_M, _D), lambda t, m: (t, 0)),
            pl.BlockSpec((1, _D, _H), lambda t, m: (m[0, t], 0, 0)),
            pl.BlockSpec((1, 1, _H), lambda t, m: (m[0, t], 0, 0)),
            pl.BlockSpec((1, _H, _D), lambda t, m: (m[0, t], 0, 0)),
            pl.BlockSpec((1, 1, _D), lambda t, m: (m[0, t], 0, 0)),
        ],
        out_specs=pl.BlockSpec((_M, _D), lambda t, m: (t, 0)),
    )
    return pl.pallas_call(
        _mlp_body,
        grid_spec=grid_spec,
        out_shape=jax.ShapeDtypeStruct((_R, _D), jnp.float32),
        compiler_params=pltpu.CompilerParams(
            dimension_semantics=("arbitrary",)),
    )(meta, xs, w1, b1, w2, b2)


@functools.lru_cache(maxsize=None)
def _make_sc_dispatch():
    per_w = _B // _NW
    mesh = plsc.VectorSubcoreMesh(core_axis_name="c", subcore_axis_name="s")

    @functools.partial(
        pl.kernel,
        mesh=mesh,
        out_type=jax.ShapeDtypeStruct((_R, _D), jnp.float32),
        scratch_types=[
            pltpu.VMEM((per_w,), jnp.int32),
            pltpu.VMEM((per_w,), jnp.int32),
            pltpu.VMEM((per_w, _D), jnp.float32),
            pltpu.SemaphoreType.DMA,
            pltpu.SemaphoreType.DMA,
        ],
    )
    def k(x_hbm, d1_hbm, d2_hbm, out_hbm, i1_v, i2_v, buf, s1, s2):
        wid = lax.axis_index("s") * 2 + lax.axis_index("c")
        base = wid * per_w
        pltpu.sync_copy(d1_hbm.at[pl.ds(base, per_w)], i1_v)
        pltpu.sync_copy(d2_hbm.at[pl.ds(base, per_w)], i2_v)
        pltpu.sync_copy(x_hbm.at[pl.ds(base, per_w)], buf)
        c1 = pltpu.async_copy(buf, out_hbm.at[i1_v], s1)
        c2 = pltpu.async_copy(buf, out_hbm.at[i2_v], s2)
        c1.wait()
        c2.wait()

    return k


@functools.lru_cache(maxsize=None)
def _make_sc_gather(n_idx, d, chunk):
    per_w = n_idx // _NW
    n_ch = per_w // chunk
    mesh = plsc.VectorSubcoreMesh(core_axis_name="c", subcore_axis_name="s")

    @functools.partial(
        pl.kernel,
        mesh=mesh,
        out_type=jax.ShapeDtypeStruct((n_idx, d), jnp.float32),
        scratch_types=[
            pltpu.VMEM((per_w,), jnp.int32),
            pltpu.VMEM((chunk, d), jnp.float32),
            pltpu.SemaphoreType.DMA,
        ],
    )
    def k(src_hbm, idx_hbm, out_hbm, idx_v, buf, sem):
        wid = lax.axis_index("s") * 2 + lax.axis_index("c")
        base = wid * per_w
        pltpu.sync_copy(idx_hbm.at[pl.ds(base, per_w)], idx_v)

        def body(i, carry):
            off = i * chunk
            pltpu.async_copy(src_hbm.at[idx_v.at[pl.ds(off, chunk)]],
                             buf, sem).wait()
            pltpu.sync_copy(buf, out_hbm.at[pl.ds(base + off, chunk)])
            return carry

        lax.fori_loop(0, n_ch, body, 0)

    return k


def _sc_gather_dispatch(src, idx):
    return _make_sc_gather(_R, _D, 64)(src, idx)


def _sc_gather_combine(src, idx):
    return _make_sc_gather(_A, _D, 64)(src, idx)


def _add_body(a_ref, w_ref, out_ref):
    out_ref[...] = (a_ref[:, 0, :] * w_ref[:, 0:1]
                    + a_ref[:, 1, :] * w_ref[:, 1:2])


def _combine_add(ys_pairs, w12):
    return pl.pallas_call(
        _add_body,
        grid=(_B // 256,),
        in_specs=[
            pl.BlockSpec((256, 2, _D), lambda i: (i, 0, 0)),
            pl.BlockSpec((256, 2), lambda i: (i, 0)),
        ],
        out_specs=pl.BlockSpec((256, _D), lambda i: (i, 0)),
        out_shape=jax.ShapeDtypeStruct((_B, _D), jnp.float32),
    )(ys_pairs, w12)


def kernel(x, Wg, bg, W1, b1, W2, b2):
    wg_pad = jnp.concatenate(
        [Wg, jnp.zeros((_D, _EPAD - _E), jnp.float32)], axis=1)
    bg_pad = jnp.concatenate(
        [bg, jnp.full((_EPAD - _E,), -1e30, jnp.float32)], axis=0)
    bg_pad = bg_pad.reshape(1, _EPAD)
    e1, e2, w1g, w2g, aux = _gate(x, wg_pad, bg_pad)

    er = jnp.arange(_E, dtype=jnp.int32)[None, :]
    oh1 = e1 == er
    oh2 = e2 == er
    on = oh1.astype(jnp.int32) + oh2.astype(jnp.int32)
    cum = jnp.cumsum(on, axis=0) - on
    counts = cum[-1] + on[-1]
    ntiles = (counts + _M - 1) // _M
    cum_tiles = jnp.cumsum(ntiles)
    pad_off = (cum_tiles - ntiles) * _M
    po_b = pad_off[None, :] + cum
    d1 = jnp.sum(jnp.where(oh1, po_b, 0), axis=1).astype(jnp.int32)
    d2 = jnp.sum(jnp.where(oh2, po_b, 0), axis=1).astype(jnp.int32)
    dest = jnp.stack([d1, d2], axis=1).reshape(-1)
    total_tiles = cum_tiles[_E - 1]
    tid = jnp.arange(_TMAX, dtype=jnp.int32)
    tclamp = jnp.minimum(tid, total_tiles - 1)
    texp = jnp.searchsorted(cum_tiles, tclamp, side="right").astype(jnp.int32)
    tvalid = (tid < total_tiles).astype(jnp.int32)
    meta = jnp.stack([texp, tvalid], axis=0)

    xs = _make_sc_dispatch()(x, d1, d2)
    ys = _grouped_mlp(meta, xs, W1.astype(jnp.bfloat16),
                      b1.reshape(_E, 1, _H),
                      W2.astype(jnp.bfloat16), b2.reshape(_E, 1, _D))

    ys_pairs = _sc_gather_combine(ys, dest)
    w12 = jnp.concatenate([w1g, w2g], axis=1)
    out = _combine_add(ys_pairs.reshape(_B, _K, _D), w12)

    return (out, aux[0, 0])


# speedup vs baseline: 1.4018x; 1.4018x over previous
import functools
import math

import jax
import jax.numpy as jnp
from jax import lax
from jax.experimental import pallas as pl
from jax.experimental.pallas import tpu as pltpu
from jax.experimental.pallas import tpu_sc as plsc

_B, _D, _H, _E, _K = 2048, 1024, 2048, 8, 2
_M = 256
_TMAX = 24
_R = _TMAX * _M
_A = _B * _K
_NW = 32
_EPAD = 128


def _gate_body(x_ref, wg_ref, bg_ref, e1_ref, e2_ref, w1_ref, w2_ref,
               aux_ref):
    s = jnp.dot(x_ref[...], wg_ref[...],
                preferred_element_type=jnp.float32) + bg_ref[...]
    col = lax.broadcasted_iota(jnp.int32, s.shape, 1)
    big = jnp.int32(2 ** 30)
    m1 = jnp.max(s, axis=1, keepdims=True)
    e1 = jnp.min(jnp.where(s >= m1, col, big), axis=1, keepdims=True)
    s2 = jnp.where(col == e1, -jnp.inf, s)
    m2 = jnp.max(s2, axis=1, keepdims=True)
    e2 = jnp.min(jnp.where(s2 >= m2, col, big), axis=1, keepdims=True)
    p = jnp.exp(m2 - m1)
    e1_ref[...] = e1
    e2_ref[...] = e2
    w1_ref[...] = 1.0 / (1.0 + p)
    w2_ref[...] = p / (1.0 + p)
    on = (jnp.where(col == e1, 1.0, 0.0) + jnp.where(col == e2, 1.0, 0.0))
    frac = jnp.sum(on, axis=0, keepdims=True) * (1.0 / (_B * _K))
    d = jnp.where(col[0:1, :] < _E, (frac - 1.0 / _E) ** 2, 0.0)
    aux_ref[0, 0] = jnp.sum(d) * (1.0 / _E)


def _gate(x, wg_pad, bg_pad):
    return pl.pallas_call(
        _gate_body,
        out_shape=(
            jax.ShapeDtypeStruct((_B, 1), jnp.int32),
            jax.ShapeDtypeStruct((_B, 1), jnp.int32),
            jax.ShapeDtypeStruct((_B, 1), jnp.float32),
            jax.ShapeDtypeStruct((_B, 1), jnp.float32),
            jax.ShapeDtypeStruct((1, 1), jnp.float32),
        ),
        out_specs=(
            pl.BlockSpec((_B, 1), lambda: (0, 0)),
            pl.BlockSpec((_B, 1), lambda: (0, 0)),
            pl.BlockSpec((_B, 1), lambda: (0, 0)),
            pl.BlockSpec((_B, 1), lambda: (0, 0)),
            pl.BlockSpec(memory_space=pltpu.SMEM),
        ),
    )(x, wg_pad, bg_pad)


def _mlp_body(meta_ref, xs_ref, w1_ref, b1_ref, w2_ref, b2_ref, out_ref):
    t = pl.program_id(0)

    @pl.when(meta_ref[1, t] == 1)
    def _():
        xg = xs_ref[...]
        h = jnp.dot(xg, w1_ref[0], preferred_element_type=jnp.float32)
        h = h + b1_ref[0]
        h = 0.5 * h * (1.0 + lax.erf(h * (1.0 / math.sqrt(2.0))))
        y = jnp.dot(h, w2_ref[0], preferred_element_type=jnp.float32)
        out_ref[...] = y + b2_ref[0]


def _grouped_mlp(meta, xs, w1, b1, w2, b2):
    grid_spec = pltpu.PrefetchScalarGridSpec(
        num_scalar_prefetch=1,
        grid=(_TMAX,),
        in_specs=[
            pl.BlockSpec((_M, _D), lambda t, m: (t, 0)),
            pl.BlockSpec((1, _D, _H), lambda t, m: (m[0, t], 0, 0)),
            pl.BlockSpec((1, 1, _H), lambda t, m: (m[0, t], 0, 0)),
            pl.BlockSpec((1, _H, _D), lambda t, m: (m[0, t], 0, 0)),
            pl.BlockSpec((1, 1, _D), lambda t, m: (m[0, t], 0, 0)),
        ],
        out_specs=pl.BlockSpec((_M, _D), lambda t, m: (t, 0)),
    )
    return pl.pallas_call(
        _mlp_body,
        grid_spec=grid_spec,
        out_shape=jax.ShapeDtypeStruct((_R, _D), jnp.float32),
        compiler_params=pltpu.CompilerParams(
            dimension_semantics=("arbitrary",)),
    )(meta, xs, w1, b1, w2, b2)


@functools.lru_cache(maxsize=None)
def _make_sc_dispatch():
    per_w = _B // _NW
    mesh = plsc.VectorSubcoreMesh(core_axis_name="c", subcore_axis_name="s")

    @functools.partial(
        pl.kernel,
        mesh=mesh,
        out_type=jax.ShapeDtypeStruct((_R, _D), jnp.float32),
        scratch_types=[
            pltpu.VMEM((per_w,), jnp.int32),
            pltpu.VMEM((per_w,), jnp.int32),
            pltpu.VMEM((per_w, _D), jnp.float32),
            pltpu.SemaphoreType.DMA,
            pltpu.SemaphoreType.DMA,
        ],
    )
    def k(x_hbm, d1_hbm, d2_hbm, out_hbm, i1_v, i2_v, buf, s1, s2):
        wid = lax.axis_index("s") * 2 + lax.axis_index("c")
        base = wid * per_w
        pltpu.sync_copy(d1_hbm.at[pl.ds(base, per_w)], i1_v)
        pltpu.sync_copy(d2_hbm.at[pl.ds(base, per_w)], i2_v)
        pltpu.sync_copy(x_hbm.at[pl.ds(base, per_w)], buf)
        c1 = pltpu.async_copy(buf, out_hbm.at[i1_v], s1)
        c2 = pltpu.async_copy(buf, out_hbm.at[i2_v], s2)
        c1.wait()
        c2.wait()

    return k


@functools.lru_cache(maxsize=None)
def _make_sc_gather(n_idx, d, chunk):
    per_w = n_idx // _NW
    n_ch = per_w // chunk
    mesh = plsc.VectorSubcoreMesh(core_axis_name="c", subcore_axis_name="s")

    @functools.partial(
        pl.kernel,
        mesh=mesh,
        out_type=jax.ShapeDtypeStruct((n_idx, d), jnp.float32),
        scratch_types=[
            pltpu.VMEM((per_w,), jnp.int32),
            pltpu.VMEM((chunk, d), jnp.float32),
            pltpu.SemaphoreType.DMA,
        ],
    )
    def k(src_hbm, idx_hbm, out_hbm, idx_v, buf, sem):
        wid = lax.axis_index("s") * 2 + lax.axis_index("c")
        base = wid * per_w
        pltpu.sync_copy(idx_hbm.at[pl.ds(base, per_w)], idx_v)

        def body(i, carry):
            off = i * chunk
            pltpu.async_copy(src_hbm.at[idx_v.at[pl.ds(off, chunk)]],
                             buf, sem).wait()
            pltpu.sync_copy(buf, out_hbm.at[pl.ds(base + off, chunk)])
            return carry

        lax.fori_loop(0, n_ch, body, 0)

    return k


def _sc_gather_dispatch(src, idx):
    return _make_sc_gather(_R, _D, 64)(src, idx)


def _sc_gather_combine(src, idx):
    return _make_sc_gather(_A, _D, 64)(src, idx)


def _add_body(a_ref, b_ref, wa_ref, wb_ref, out_ref):
    out_ref[...] = a_ref[...] * wa_ref[...] + b_ref[...] * wb_ref[...]


_NBLK = _B // 256


def _combine_add(ys_ab, w1g, w2g):
    return pl.pallas_call(
        _add_body,
        grid=(_NBLK,),
        in_specs=[
            pl.BlockSpec((256, _D), lambda i: (i, 0)),
            pl.BlockSpec((256, _D), lambda i: (i + _NBLK, 0)),
            pl.BlockSpec((256, 1), lambda i: (i, 0)),
            pl.BlockSpec((256, 1), lambda i: (i, 0)),
        ],
        out_specs=pl.BlockSpec((256, _D), lambda i: (i, 0)),
        out_shape=jax.ShapeDtypeStruct((_B, _D), jnp.float32),
    )(ys_ab, ys_ab, w1g, w2g)


def kernel(x, Wg, bg, W1, b1, W2, b2):
    wg_pad = jnp.concatenate(
        [Wg, jnp.zeros((_D, _EPAD - _E), jnp.float32)], axis=1)
    bg_pad = jnp.concatenate(
        [bg, jnp.full((_EPAD - _E,), -1e30, jnp.float32)], axis=0)
    bg_pad = bg_pad.reshape(1, _EPAD)
    e1, e2, w1g, w2g, aux = _gate(x, wg_pad, bg_pad)

    er = jnp.arange(_E, dtype=jnp.int32)[None, :]
    oh1 = e1 == er
    oh2 = e2 == er
    on = oh1.astype(jnp.int32) + oh2.astype(jnp.int32)
    cum = jnp.cumsum(on, axis=0) - on
    counts = cum[-1] + on[-1]
    ntiles = (counts + _M - 1) // _M
    cum_tiles = jnp.cumsum(ntiles)
    pad_off = (cum_tiles - ntiles) * _M
    po_b = pad_off[None, :] + cum
    d1 = jnp.sum(jnp.where(oh1, po_b, 0), axis=1).astype(jnp.int32)
    d2 = jnp.sum(jnp.where(oh2, po_b, 0), axis=1).astype(jnp.int32)
    dest = jnp.concatenate([d1, d2], axis=0)
    total_tiles = cum_tiles[_E - 1]
    tid = jnp.arange(_TMAX, dtype=jnp.int32)
    tclamp = jnp.minimum(tid, total_tiles - 1)
    texp = jnp.searchsorted(cum_tiles, tclamp, side="right").astype(jnp.int32)
    tvalid = (tid < total_tiles).astype(jnp.int32)
    meta = jnp.stack([texp, tvalid], axis=0)

    xs = _make_sc_dispatch()(x, d1, d2)
    ys = _grouped_mlp(meta, xs, W1, b1.reshape(_E, 1, _H),
                      W2, b2.reshape(_E, 1, _D))

    ys_ab = _sc_gather_combine(ys, dest)
    out = _combine_add(ys_ab, w1g, w2g)

    return (out, aux[0, 0])


# speedup vs baseline: 1.5566x; 1.1104x over previous
import functools
import math

import jax
import jax.numpy as jnp
from jax import lax
from jax.experimental import pallas as pl
from jax.experimental.pallas import tpu as pltpu
from jax.experimental.pallas import tpu_sc as plsc

_B, _D, _H, _E, _K = 2048, 1024, 2048, 8, 2
_M = 256
_TMAX = 24
_R = _TMAX * _M
_A = _B * _K
_NW = 32
_EPAD = 128


def _gate_body(x_ref, wg_ref, bg_ref, e1_ref, e2_ref, w1_ref, w2_ref,
               aux_ref):
    s = jnp.dot(x_ref[...], wg_ref[...],
                preferred_element_type=jnp.float32) + bg_ref[...]
    col = lax.broadcasted_iota(jnp.int32, s.shape, 1)
    big = jnp.int32(2 ** 30)
    m1 = jnp.max(s, axis=1, keepdims=True)
    e1 = jnp.min(jnp.where(s >= m1, col, big), axis=1, keepdims=True)
    s2 = jnp.where(col == e1, -jnp.inf, s)
    m2 = jnp.max(s2, axis=1, keepdims=True)
    e2 = jnp.min(jnp.where(s2 >= m2, col, big), axis=1, keepdims=True)
    p = jnp.exp(m2 - m1)
    e1_ref[...] = e1
    e2_ref[...] = e2
    w1_ref[...] = 1.0 / (1.0 + p)
    w2_ref[...] = p / (1.0 + p)
    on = (jnp.where(col == e1, 1.0, 0.0) + jnp.where(col == e2, 1.0, 0.0))
    frac = jnp.sum(on, axis=0, keepdims=True) * (1.0 / (_B * _K))
    d = jnp.where(col[0:1, :] < _E, (frac - 1.0 / _E) ** 2, 0.0)
    aux_ref[0, 0] = jnp.sum(d) * (1.0 / _E)


def _gate(x, wg_pad, bg_pad):
    return pl.pallas_call(
        _gate_body,
        out_shape=(
            jax.ShapeDtypeStruct((_B, 1), jnp.int32),
            jax.ShapeDtypeStruct((_B, 1), jnp.int32),
            jax.ShapeDtypeStruct((_B, 1), jnp.float32),
            jax.ShapeDtypeStruct((_B, 1), jnp.float32),
            jax.ShapeDtypeStruct((1, 1), jnp.float32),
        ),
        out_specs=(
            pl.BlockSpec((_B, 1), lambda: (0, 0)),
            pl.BlockSpec((_B, 1), lambda: (0, 0)),
            pl.BlockSpec((_B, 1), lambda: (0, 0)),
            pl.BlockSpec((_B, 1), lambda: (0, 0)),
            pl.BlockSpec(memory_space=pltpu.SMEM),
        ),
    )(x, wg_pad, bg_pad)


def _mlp_body(meta_ref, xs_ref, b1_ref, b2_ref, w1_hbm, w2_hbm, out_ref,
              w1b, w2b, s1, s2):
    t = pl.program_id(0)
    e = meta_ref[0, t]
    valid = meta_ref[1, t]
    slot = meta_ref[2, t]
    first = meta_ref[3, t]
    nxt = meta_ref[4, t]
    has_next = meta_ref[5, t]

    @pl.when(t == 0)
    def _():
        pltpu.make_async_copy(w1_hbm.at[e], w1b.at[0], s1.at[0]).start()
        pltpu.make_async_copy(w2_hbm.at[e], w2b.at[0], s2.at[0]).start()

    def _boundary(cs, ns):
        pltpu.make_async_copy(w1_hbm.at[e], w1b.at[cs], s1.at[cs]).wait()
        pltpu.make_async_copy(w2_hbm.at[e], w2b.at[cs], s2.at[cs]).wait()

        @pl.when(has_next == 1)
        def _():
            pltpu.make_async_copy(w1_hbm.at[nxt], w1b.at[ns],
                                  s1.at[ns]).start()
            pltpu.make_async_copy(w2_hbm.at[nxt], w2b.at[ns],
                                  s2.at[ns]).start()

    @pl.when(jnp.logical_and(first == 1, slot == 0))
    def _():
        _boundary(0, 1)

    @pl.when(jnp.logical_and(first == 1, slot == 1))
    def _():
        _boundary(1, 0)

    def _compute(w1v, w2v):
        xg = xs_ref[...]
        h = jnp.dot(xg, w1v, preferred_element_type=jnp.float32)
        h = h + b1_ref[0]
        h = 0.5 * h * (1.0 + lax.erf(h * (1.0 / math.sqrt(2.0))))
        y = jnp.dot(h, w2v, preferred_element_type=jnp.float32)
        out_ref[...] = y + b2_ref[0]

    @pl.when(jnp.logical_and(valid == 1, slot == 0))
    def _():
        _compute(w1b[0], w2b[0])

    @pl.when(jnp.logical_and(valid == 1, slot == 1))
    def _():
        _compute(w1b[1], w2b[1])


def _grouped_mlp(meta, xs, w1, b1, w2, b2):
    grid_spec = pltpu.PrefetchScalarGridSpec(
        num_scalar_prefetch=1,
        grid=(_TMAX,),
        in_specs=[
            pl.BlockSpec((_M, _D), lambda t, m: (t, 0)),
            pl.BlockSpec((1, 1, _H), lambda t, m: (m[0, t], 0, 0)),
            pl.BlockSpec((1, 1, _D), lambda t, m: (m[0, t], 0, 0)),
            pl.BlockSpec(memory_space=pl.ANY),
            pl.BlockSpec(memory_space=pl.ANY),
        ],
        out_specs=pl.BlockSpec((_M, _D), lambda t, m: (t, 0)),
        scratch_shapes=[
            pltpu.VMEM((2, _D, _H), jnp.float32),
            pltpu.VMEM((2, _H, _D), jnp.float32),
            pltpu.SemaphoreType.DMA((2,)),
            pltpu.SemaphoreType.DMA((2,)),
        ],
    )
    return pl.pallas_call(
        _mlp_body,
        grid_spec=grid_spec,
        out_shape=jax.ShapeDtypeStruct((_R, _D), jnp.float32),
        compiler_params=pltpu.CompilerParams(
            dimension_semantics=("arbitrary",)),
    )(meta, xs, b1, b2, w1, w2)


@functools.lru_cache(maxsize=None)
def _make_sc_dispatch():
    per_w = _B // _NW
    mesh = plsc.VectorSubcoreMesh(core_axis_name="c", subcore_axis_name="s")

    @functools.partial(
        pl.kernel,
        mesh=mesh,
        out_type=jax.ShapeDtypeStruct((_R, _D), jnp.float32),
        scratch_types=[
            pltpu.VMEM((per_w,), jnp.int32),
            pltpu.VMEM((per_w,), jnp.int32),
            pltpu.VMEM((per_w, _D), jnp.float32),
            pltpu.SemaphoreType.DMA,
            pltpu.SemaphoreType.DMA,
        ],
    )
    def k(x_hbm, d1_hbm, d2_hbm, out_hbm, i1_v, i2_v, buf, s1, s2):
        wid = lax.axis_index("s") * 2 + lax.axis_index("c")
        base = wid * per_w
        pltpu.sync_copy(d1_hbm.at[pl.ds(base, per_w)], i1_v)
        pltpu.sync_copy(d2_hbm.at[pl.ds(base, per_w)], i2_v)
        pltpu.sync_copy(x_hbm.at[pl.ds(base, per_w)], buf)
        c1 = pltpu.async_copy(buf, out_hbm.at[i1_v], s1)
        c2 = pltpu.async_copy(buf, out_hbm.at[i2_v], s2)
        c1.wait()
        c2.wait()

    return k


@functools.lru_cache(maxsize=None)
def _make_sc_gather(n_idx, d, chunk):
    per_w = n_idx // _NW
    n_ch = per_w // chunk
    mesh = plsc.VectorSubcoreMesh(core_axis_name="c", subcore_axis_name="s")

    @functools.partial(
        pl.kernel,
        mesh=mesh,
        out_type=jax.ShapeDtypeStruct((n_idx, d), jnp.float32),
        scratch_types=[
            pltpu.VMEM((per_w,), jnp.int32),
            pltpu.VMEM((chunk, d), jnp.float32),
            pltpu.SemaphoreType.DMA,
        ],
    )
    def k(src_hbm, idx_hbm, out_hbm, idx_v, buf, sem):
        wid = lax.axis_index("s") * 2 + lax.axis_index("c")
        base = wid * per_w
        pltpu.sync_copy(idx_hbm.at[pl.ds(base, per_w)], idx_v)

        def body(i, carry):
            off = i * chunk
            pltpu.async_copy(src_hbm.at[idx_v.at[pl.ds(off, chunk)]],
                             buf, sem).wait()
            pltpu.sync_copy(buf, out_hbm.at[pl.ds(base + off, chunk)])
            return carry

        lax.fori_loop(0, n_ch, body, 0)

    return k


def _sc_gather_dispatch(src, idx):
    return _make_sc_gather(_R, _D, 64)(src, idx)


def _sc_gather_combine(src, idx):
    return _make_sc_gather(_A, _D, 64)(src, idx)


def _add_body(a_ref, b_ref, wa_ref, wb_ref, out_ref):
    out_ref[...] = a_ref[...] * wa_ref[...] + b_ref[...] * wb_ref[...]


_NBLK = _B // 256


def _combine_add(ys_ab, w1g, w2g):
    return pl.pallas_call(
        _add_body,
        grid=(_NBLK,),
        in_specs=[
            pl.BlockSpec((256, _D), lambda i: (i, 0)),
            pl.BlockSpec((256, _D), lambda i: (i + _NBLK, 0)),
            pl.BlockSpec((256, 1), lambda i: (i, 0)),
            pl.BlockSpec((256, 1), lambda i: (i, 0)),
        ],
        out_specs=pl.BlockSpec((256, _D), lambda i: (i, 0)),
        out_shape=jax.ShapeDtypeStruct((_B, _D), jnp.float32),
    )(ys_ab, ys_ab, w1g, w2g)


def kernel(x, Wg, bg, W1, b1, W2, b2):
    wg_pad = jnp.concatenate(
        [Wg, jnp.zeros((_D, _EPAD - _E), jnp.float32)], axis=1)
    bg_pad = jnp.concatenate(
        [bg, jnp.full((_EPAD - _E,), -1e30, jnp.float32)], axis=0)
    bg_pad = bg_pad.reshape(1, _EPAD)
    e1, e2, w1g, w2g, aux = _gate(x, wg_pad, bg_pad)

    er = jnp.arange(_E, dtype=jnp.int32)[None, :]
    oh1 = e1 == er
    oh2 = e2 == er
    on = oh1.astype(jnp.int32) + oh2.astype(jnp.int32)
    cum = jnp.cumsum(on, axis=0) - on
    counts = cum[-1] + on[-1]
    ntiles = (counts + _M - 1) // _M
    cum_tiles = jnp.cumsum(ntiles)
    pad_off = (cum_tiles - ntiles) * _M
    po_b = pad_off[None, :] + cum
    d1 = jnp.sum(jnp.where(oh1, po_b, 0), axis=1).astype(jnp.int32)
    d2 = jnp.sum(jnp.where(oh2, po_b, 0), axis=1).astype(jnp.int32)
    dest = jnp.concatenate([d1, d2], axis=0)
    total_tiles = cum_tiles[_E - 1]
    tid = jnp.arange(_TMAX, dtype=jnp.int32)
    tclamp = jnp.minimum(tid, total_tiles - 1)
    texp = jnp.searchsorted(cum_tiles, tclamp, side="right").astype(jnp.int32)
    tvalid = (tid < total_tiles).astype(jnp.int32)
    nz = (ntiles > 0).astype(jnp.int32)
    grpidx = jnp.cumsum(nz) - nz
    eq = (texp[:, None] == jnp.arange(_E, dtype=jnp.int32)[None, :])
    slot_t = jnp.sum(jnp.where(eq, grpidx[None, :], 0), axis=1) % 2
    nf = jnp.sum(jnp.where(eq, cum_tiles[None, :], 0), axis=1)
    has_next = (nf < total_tiles).astype(jnp.int32)
    next_e = jnp.searchsorted(
        cum_tiles, jnp.minimum(nf, total_tiles - 1),
        side="right").astype(jnp.int32)
    prev_exp = jnp.concatenate([jnp.full((1,), -1, jnp.int32), texp[:-1]])
    first_t = (tvalid * (texp != prev_exp)).astype(jnp.int32)
    meta = jnp.stack([texp, tvalid, slot_t, first_t, next_e, has_next],
                     axis=0)

    xs = _make_sc_dispatch()(x, d1, d2)
    ys = _grouped_mlp(meta, xs, W1, b1.reshape(_E, 1, _H),
                      W2, b2.reshape(_E, 1, _D))

    ys_ab = _sc_gather_combine(ys, dest)
    out = _combine_add(ys_ab, w1g, w2g)

    return (out, aux[0, 0])


# speedup vs baseline: 1.5957x; 1.0251x over previous
import functools
import math

import jax
import jax.numpy as jnp
from jax import lax
from jax.experimental import pallas as pl
from jax.experimental.pallas import tpu as pltpu
from jax.experimental.pallas import tpu_sc as plsc

_B, _D, _H, _E, _K = 2048, 1024, 2048, 8, 2
_M = 256
_TMAX = 24
_R = _TMAX * _M
_A = _B * _K
_NW = 32
_EPAD = 128


def _gate_body(x_ref, wg_ref, bg_ref, d1_ref, d2_ref, w1_ref, w2_ref,
               cnt_ref, aux_ref):
    s = jnp.dot(x_ref[...], wg_ref[...],
                preferred_element_type=jnp.float32) + bg_ref[...]
    col = lax.broadcasted_iota(jnp.int32, s.shape, 1)
    big = jnp.int32(2 ** 30)
    m1 = jnp.max(s, axis=1, keepdims=True)
    e1 = jnp.min(jnp.where(s >= m1, col, big), axis=1, keepdims=True)
    s2 = jnp.where(col == e1, -jnp.inf, s)
    m2 = jnp.max(s2, axis=1, keepdims=True)
    e2 = jnp.min(jnp.where(s2 >= m2, col, big), axis=1, keepdims=True)
    p = jnp.exp(m2 - m1)
    w1_ref[...] = 1.0 / (1.0 + p)
    w2_ref[...] = p / (1.0 + p)
    oh1 = col == e1
    oh2 = col == e2
    on = jnp.where(oh1, 1.0, 0.0) + jnp.where(oh2, 1.0, 0.0)
    acc = on
    k = 1
    while k < _B:
        acc = acc + jnp.concatenate(
            [jnp.zeros((k, _E), jnp.float32), acc[:-k, :]], axis=0)
        k *= 2
    cum = acc - on
    counts = acc[_B - 1:_B, :]
    ntiles = jnp.floor((counts + (_M - 1)) * (1.0 / _M))
    ct = ntiles
    k = 1
    while k < _E:
        ct = ct + jnp.concatenate(
            [jnp.zeros((1, k), jnp.float32), ct[:, :-k]], axis=1)
        k *= 2
    pad_off = (ct - ntiles) * _M
    po_b = pad_off + cum
    d1_ref[...] = jnp.sum(jnp.where(oh1, po_b, 0.0), axis=1,
                          keepdims=True).astype(jnp.int32)
    d2_ref[...] = jnp.sum(jnp.where(oh2, po_b, 0.0), axis=1,
                          keepdims=True).astype(jnp.int32)
    cnt_ref[...] = counts
    frac = counts * (1.0 / (_B * _K))
    aux_ref[0, 0] = jnp.sum((frac - 1.0 / _E) ** 2) * (1.0 / _E)


def _gate(x, wg, bg_row):
    return pl.pallas_call(
        _gate_body,
        out_shape=(
            jax.ShapeDtypeStruct((_B, 1), jnp.int32),
            jax.ShapeDtypeStruct((_B, 1), jnp.int32),
            jax.ShapeDtypeStruct((_B, 1), jnp.float32),
            jax.ShapeDtypeStruct((_B, 1), jnp.float32),
            jax.ShapeDtypeStruct((1, _E), jnp.float32),
            jax.ShapeDtypeStruct((1, 1), jnp.float32),
        ),
        out_specs=(
            pl.BlockSpec((_B, 1), lambda: (0, 0)),
            pl.BlockSpec((_B, 1), lambda: (0, 0)),
            pl.BlockSpec((_B, 1), lambda: (0, 0)),
            pl.BlockSpec((_B, 1), lambda: (0, 0)),
            pl.BlockSpec((1, _E), lambda: (0, 0)),
            pl.BlockSpec(memory_space=pltpu.SMEM),
        ),
    )(x, wg, bg_row)


def _mlp_body(meta_ref, xs_ref, b1_ref, b2_ref, w1_hbm, w2_hbm, out_ref,
              w1b, w2b, s1, s2):
    t = pl.program_id(0)
    e = meta_ref[0, t]
    valid = meta_ref[1, t]
    slot = meta_ref[2, t]
    first = meta_ref[3, t]
    nxt = meta_ref[4, t]
    has_next = meta_ref[5, t]

    @pl.when(t == 0)
    def _():
        pltpu.make_async_copy(w1_hbm.at[e], w1b.at[0], s1.at[0]).start()
        pltpu.make_async_copy(w2_hbm.at[e], w2b.at[0], s2.at[0]).start()

    def _boundary(cs, ns):
        pltpu.make_async_copy(w1_hbm.at[e], w1b.at[cs], s1.at[cs]).wait()
        pltpu.make_async_copy(w2_hbm.at[e], w2b.at[cs], s2.at[cs]).wait()

        @pl.when(has_next == 1)
        def _():
            pltpu.make_async_copy(w1_hbm.at[nxt], w1b.at[ns],
                                  s1.at[ns]).start()
            pltpu.make_async_copy(w2_hbm.at[nxt], w2b.at[ns],
                                  s2.at[ns]).start()

    @pl.when(jnp.logical_and(first == 1, slot == 0))
    def _():
        _boundary(0, 1)

    @pl.when(jnp.logical_and(first == 1, slot == 1))
    def _():
        _boundary(1, 0)

    def _compute(w1v, w2v):
        xg = xs_ref[...]
        h = jnp.dot(xg, w1v, preferred_element_type=jnp.float32)
        h = h + b1_ref[0]
        h = 0.5 * h * (1.0 + lax.erf(h * (1.0 / math.sqrt(2.0))))
        y = jnp.dot(h, w2v, preferred_element_type=jnp.float32)
        out_ref[...] = y + b2_ref[0]

    @pl.when(jnp.logical_and(valid == 1, slot == 0))
    def _():
        _compute(w1b[0], w2b[0])

    @pl.when(jnp.logical_and(valid == 1, slot == 1))
    def _():
        _compute(w1b[1], w2b[1])


def _grouped_mlp(meta, xs, w1, b1, w2, b2):
    grid_spec = pltpu.PrefetchScalarGridSpec(
        num_scalar_prefetch=1,
        grid=(_TMAX,),
        in_specs=[
            pl.BlockSpec((_M, _D), lambda t, m: (t, 0)),
            pl.BlockSpec((1, 1, _H), lambda t, m: (m[0, t], 0, 0)),
            pl.BlockSpec((1, 1, _D), lambda t, m: (m[0, t], 0, 0)),
            pl.BlockSpec(memory_space=pl.ANY),
            pl.BlockSpec(memory_space=pl.ANY),
        ],
        out_specs=pl.BlockSpec((_M, _D), lambda t, m: (t, 0)),
        scratch_shapes=[
            pltpu.VMEM((2, _D, _H), jnp.float32),
            pltpu.VMEM((2, _H, _D), jnp.float32),
            pltpu.SemaphoreType.DMA((2,)),
            pltpu.SemaphoreType.DMA((2,)),
        ],
    )
    return pl.pallas_call(
        _mlp_body,
        grid_spec=grid_spec,
        out_shape=jax.ShapeDtypeStruct((_R, _D), jnp.float32),
        compiler_params=pltpu.CompilerParams(
            dimension_semantics=("arbitrary",)),
    )(meta, xs, b1, b2, w1, w2)


@functools.lru_cache(maxsize=None)
def _make_sc_dispatch():
    per_w = _B // _NW
    mesh = plsc.VectorSubcoreMesh(core_axis_name="c", subcore_axis_name="s")

    @functools.partial(
        pl.kernel,
        mesh=mesh,
        out_type=jax.ShapeDtypeStruct((_R, _D), jnp.float32),
        scratch_types=[
            pltpu.VMEM((per_w,), jnp.int32),
            pltpu.VMEM((per_w,), jnp.int32),
            pltpu.VMEM((per_w, _D), jnp.float32),
            pltpu.SemaphoreType.DMA,
            pltpu.SemaphoreType.DMA,
        ],
    )
    def k(x_hbm, d1_hbm, d2_hbm, out_hbm, i1_v, i2_v, buf, s1, s2):
        wid = lax.axis_index("s") * 2 + lax.axis_index("c")
        base = wid * per_w
        pltpu.sync_copy(d1_hbm.at[pl.ds(base, per_w)], i1_v)
        pltpu.sync_copy(d2_hbm.at[pl.ds(base, per_w)], i2_v)
        pltpu.sync_copy(x_hbm.at[pl.ds(base, per_w)], buf)
        c1 = pltpu.async_copy(buf, out_hbm.at[i1_v], s1)
        c2 = pltpu.async_copy(buf, out_hbm.at[i2_v], s2)
        c1.wait()
        c2.wait()

    return k


@functools.lru_cache(maxsize=None)
def _make_sc_gather(n_idx, d, chunk):
    per_w = n_idx // _NW
    n_ch = per_w // chunk
    mesh = plsc.VectorSubcoreMesh(core_axis_name="c", subcore_axis_name="s")

    @functools.partial(
        pl.kernel,
        mesh=mesh,
        out_type=jax.ShapeDtypeStruct((n_idx, d), jnp.float32),
        scratch_types=[
            pltpu.VMEM((per_w,), jnp.int32),
            pltpu.VMEM((chunk, d), jnp.float32),
            pltpu.SemaphoreType.DMA,
        ],
    )
    def k(src_hbm, idx_hbm, out_hbm, idx_v, buf, sem):
        wid = lax.axis_index("s") * 2 + lax.axis_index("c")
        base = wid * per_w
        pltpu.sync_copy(idx_hbm.at[pl.ds(base, per_w)], idx_v)

        def body(i, carry):
            off = i * chunk
            pltpu.async_copy(src_hbm.at[idx_v.at[pl.ds(off, chunk)]],
                             buf, sem).wait()
            pltpu.sync_copy(buf, out_hbm.at[pl.ds(base + off, chunk)])
            return carry

        lax.fori_loop(0, n_ch, body, 0)

    return k


def _sc_gather_dispatch(src, idx):
    return _make_sc_gather(_R, _D, 64)(src, idx)


def _sc_gather_combine(src, idx):
    return _make_sc_gather(_A, _D, 64)(src, idx)


def _add_body(a_ref, b_ref, wa_ref, wb_ref, out_ref):
    out_ref[...] = a_ref[...] * wa_ref[...] + b_ref[...] * wb_ref[...]


_NBLK = _B // 256


def _combine_add(ys_ab, w1g, w2g):
    return pl.pallas_call(
        _add_body,
        grid=(_NBLK,),
        in_specs=[
            pl.BlockSpec((256, _D), lambda i: (i, 0)),
            pl.BlockSpec((256, _D), lambda i: (i + _NBLK, 0)),
            pl.BlockSpec((256, 1), lambda i: (i, 0)),
            pl.BlockSpec((256, 1), lambda i: (i, 0)),
        ],
        out_specs=pl.BlockSpec((256, _D), lambda i: (i, 0)),
        out_shape=jax.ShapeDtypeStruct((_B, _D), jnp.float32),
    )(ys_ab, ys_ab, w1g, w2g)


def kernel(x, Wg, bg, W1, b1, W2, b2):
    d1c, d2c, w1g, w2g, counts_f, aux = _gate(x, Wg, bg.reshape(1, _E))
    d1 = d1c[:, 0]
    d2 = d2c[:, 0]
    dest = jnp.concatenate([d1, d2], axis=0)

    counts = counts_f[0].astype(jnp.int32)
    ntiles = (counts + _M - 1) // _M
    cum_tiles = jnp.cumsum(ntiles)
    total_tiles = cum_tiles[_E - 1]
    tid = jnp.arange(_TMAX, dtype=jnp.int32)
    tclamp = jnp.minimum(tid, total_tiles - 1)
    texp = jnp.searchsorted(cum_tiles, tclamp, side="right").astype(jnp.int32)
    tvalid = (tid < total_tiles).astype(jnp.int32)
    nz = (ntiles > 0).astype(jnp.int32)
    grpidx = jnp.cumsum(nz) - nz
    eq = (texp[:, None] == jnp.arange(_E, dtype=jnp.int32)[None, :])
    slot_t = jnp.sum(jnp.where(eq, grpidx[None, :], 0), axis=1) % 2
    nf = jnp.sum(jnp.where(eq, cum_tiles[None, :], 0), axis=1)
    has_next = (nf < total_tiles).astype(jnp.int32)
    next_e = jnp.searchsorted(
        cum_tiles, jnp.minimum(nf, total_tiles - 1),
        side="right").astype(jnp.int32)
    prev_exp = jnp.concatenate([jnp.full((1,), -1, jnp.int32), texp[:-1]])
    first_t = (tvalid * (texp != prev_exp)).astype(jnp.int32)
    meta = jnp.stack([texp, tvalid, slot_t, first_t, next_e, has_next],
                     axis=0)

    xs = _make_sc_dispatch()(x, d1, d2)
    ys = _grouped_mlp(meta, xs, W1, b1.reshape(_E, 1, _H),
                      W2, b2.reshape(_E, 1, _D))

    ys_ab = _sc_gather_combine(ys, dest)
    out = _combine_add(ys_ab, w1g, w2g)

    return (out, aux[0, 0])


# speedup vs baseline: 1.7295x; 1.0838x over previous
import functools
import math

import jax
import jax.numpy as jnp
from jax import lax
from jax.experimental import pallas as pl
from jax.experimental.pallas import tpu as pltpu
from jax.experimental.pallas import tpu_sc as plsc

_B, _D, _H, _E, _K = 2048, 1024, 2048, 8, 2
_M = 256
_TMAX = 24
_R = _TMAX * _M
_A = _B * _K
_NW = 32
_EPAD = 128


def _gate_body(x_ref, wg_ref, bg_ref, d1_ref, d2_ref, w1_ref, w2_ref,
               cnt_ref, aux_ref):
    s = jnp.dot(x_ref[...], wg_ref[...],
                preferred_element_type=jnp.float32) + bg_ref[...]
    col = lax.broadcasted_iota(jnp.int32, s.shape, 1)
    big = jnp.int32(2 ** 30)
    m1 = jnp.max(s, axis=1, keepdims=True)
    e1 = jnp.min(jnp.where(s >= m1, col, big), axis=1, keepdims=True)
    s2 = jnp.where(col == e1, -jnp.inf, s)
    m2 = jnp.max(s2, axis=1, keepdims=True)
    e2 = jnp.min(jnp.where(s2 >= m2, col, big), axis=1, keepdims=True)
    p = jnp.exp(m2 - m1)
    w1_ref[...] = 1.0 / (1.0 + p)
    w2_ref[...] = p / (1.0 + p)
    oh1 = col == e1
    oh2 = col == e2
    on = jnp.where(oh1, 1.0, 0.0) + jnp.where(oh2, 1.0, 0.0)
    acc = on
    k = 1
    while k < _B:
        acc = acc + jnp.concatenate(
            [jnp.zeros((k, _E), jnp.float32), acc[:-k, :]], axis=0)
        k *= 2
    cum = acc - on
    counts = acc[_B - 1:_B, :]
    ntiles = jnp.floor((counts + (_M - 1)) * (1.0 / _M))
    ct = ntiles
    k = 1
    while k < _E:
        ct = ct + jnp.concatenate(
            [jnp.zeros((1, k), jnp.float32), ct[:, :-k]], axis=1)
        k *= 2
    pad_off = (ct - ntiles) * _M
    po_b = pad_off + cum
    d1_ref[...] = jnp.sum(jnp.where(oh1, po_b, 0.0), axis=1,
                          keepdims=True).astype(jnp.int32)
    d2_ref[...] = jnp.sum(jnp.where(oh2, po_b, 0.0), axis=1,
                          keepdims=True).astype(jnp.int32)
    cnt_ref[...] = counts
    frac = counts * (1.0 / (_B * _K))
    aux_ref[0, 0] = jnp.sum((frac - 1.0 / _E) ** 2) * (1.0 / _E)


def _gate(x, wg, bg_row):
    return pl.pallas_call(
        _gate_body,
        out_shape=(
            jax.ShapeDtypeStruct((_B, 1), jnp.int32),
            jax.ShapeDtypeStruct((_B, 1), jnp.int32),
            jax.ShapeDtypeStruct((_B, 1), jnp.float32),
            jax.ShapeDtypeStruct((_B, 1), jnp.float32),
            jax.ShapeDtypeStruct((1, _E), jnp.float32),
            jax.ShapeDtypeStruct((1, 1), jnp.float32),
        ),
        out_specs=(
            pl.BlockSpec((_B, 1), lambda: (0, 0)),
            pl.BlockSpec((_B, 1), lambda: (0, 0)),
            pl.BlockSpec((_B, 1), lambda: (0, 0)),
            pl.BlockSpec((_B, 1), lambda: (0, 0)),
            pl.BlockSpec((1, _E), lambda: (0, 0)),
            pl.BlockSpec(memory_space=pltpu.SMEM),
        ),
    )(x, wg, bg_row)


def _mlp_body(meta_ref, xs_ref, b1_ref, b2_ref, w1_hbm, w2_hbm, out_ref,
              w1b, w2b, s1, s2):
    t = pl.program_id(0)
    e = meta_ref[0, t]
    valid = meta_ref[1, t]
    slot = meta_ref[2, t]
    first = meta_ref[3, t]
    nxt = meta_ref[4, t]
    has_next = meta_ref[5, t]

    @pl.when(t == 0)
    def _():
        pltpu.make_async_copy(w1_hbm.at[e], w1b.at[0], s1.at[0]).start()
        pltpu.make_async_copy(w2_hbm.at[e], w2b.at[0], s2.at[0]).start()

    def _boundary(cs, ns):
        pltpu.make_async_copy(w1_hbm.at[e], w1b.at[cs], s1.at[cs]).wait()
        pltpu.make_async_copy(w2_hbm.at[e], w2b.at[cs], s2.at[cs]).wait()

        @pl.when(has_next == 1)
        def _():
            pltpu.make_async_copy(w1_hbm.at[nxt], w1b.at[ns],
                                  s1.at[ns]).start()
            pltpu.make_async_copy(w2_hbm.at[nxt], w2b.at[ns],
                                  s2.at[ns]).start()

    @pl.when(jnp.logical_and(first == 1, slot == 0))
    def _():
        _boundary(0, 1)

    @pl.when(jnp.logical_and(first == 1, slot == 1))
    def _():
        _boundary(1, 0)

    def _compute(w1v, w2v):
        xg = xs_ref[...]
        h = jnp.dot(xg, w1v, preferred_element_type=jnp.float32)
        h = h + b1_ref[0]
        h = 0.5 * h * (1.0 + lax.erf(h * (1.0 / math.sqrt(2.0))))
        y = jnp.dot(h, w2v, preferred_element_type=jnp.float32)
        y = y + b2_ref[0]
        out_ref[...] = pltpu.pack_elementwise(
            [y[:, :_D // 2], y[:, _D // 2:]], packed_dtype=jnp.bfloat16)

    @pl.when(jnp.logical_and(valid == 1, slot == 0))
    def _():
        _compute(w1b[0], w2b[0])

    @pl.when(jnp.logical_and(valid == 1, slot == 1))
    def _():
        _compute(w1b[1], w2b[1])


def _grouped_mlp(meta, xs, w1, b1, w2, b2):
    grid_spec = pltpu.PrefetchScalarGridSpec(
        num_scalar_prefetch=1,
        grid=(_TMAX,),
        in_specs=[
            pl.BlockSpec((_M, _D), lambda t, m: (t, 0)),
            pl.BlockSpec((1, 1, _H), lambda t, m: (m[0, t], 0, 0)),
            pl.BlockSpec((1, 1, _D), lambda t, m: (m[0, t], 0, 0)),
            pl.BlockSpec(memory_space=pl.ANY),
            pl.BlockSpec(memory_space=pl.ANY),
        ],
        out_specs=pl.BlockSpec((_M, _D // 2), lambda t, m: (t, 0)),
        scratch_shapes=[
            pltpu.VMEM((2, _D, _H), jnp.float32),
            pltpu.VMEM((2, _H, _D), jnp.float32),
            pltpu.SemaphoreType.DMA((2,)),
            pltpu.SemaphoreType.DMA((2,)),
        ],
    )
    return pl.pallas_call(
        _mlp_body,
        grid_spec=grid_spec,
        out_shape=jax.ShapeDtypeStruct((_R, _D // 2), jnp.int32),
        compiler_params=pltpu.CompilerParams(
            dimension_semantics=("arbitrary",)),
    )(meta, xs, b1, b2, w1, w2)


@functools.lru_cache(maxsize=None)
def _make_sc_dispatch():
    per_w = _B // _NW
    mesh = plsc.VectorSubcoreMesh(core_axis_name="c", subcore_axis_name="s")

    @functools.partial(
        pl.kernel,
        mesh=mesh,
        out_type=jax.ShapeDtypeStruct((_R, _D), jnp.float32),
        scratch_types=[
            pltpu.VMEM((per_w,), jnp.int32),
            pltpu.VMEM((per_w,), jnp.int32),
            pltpu.VMEM((per_w, _D), jnp.float32),
            pltpu.SemaphoreType.DMA,
            pltpu.SemaphoreType.DMA,
        ],
    )
    def k(x_hbm, d1_hbm, d2_hbm, out_hbm, i1_v, i2_v, buf, s1, s2):
        wid = lax.axis_index("s") * 2 + lax.axis_index("c")
        base = wid * per_w
        pltpu.sync_copy(d1_hbm.at[pl.ds(base, per_w)], i1_v)
        pltpu.sync_copy(d2_hbm.at[pl.ds(base, per_w)], i2_v)
        pltpu.sync_copy(x_hbm.at[pl.ds(base, per_w)], buf)
        c1 = pltpu.async_copy(buf, out_hbm.at[i1_v], s1)
        c2 = pltpu.async_copy(buf, out_hbm.at[i2_v], s2)
        c1.wait()
        c2.wait()

    return k


@functools.lru_cache(maxsize=None)
def _make_sc_gather(n_idx, d, chunk, dtype):
    per_w = n_idx // _NW
    n_ch = per_w // chunk
    mesh = plsc.VectorSubcoreMesh(core_axis_name="c", subcore_axis_name="s")

    @functools.partial(
        pl.kernel,
        mesh=mesh,
        out_type=jax.ShapeDtypeStruct((n_idx, d), dtype),
        scratch_types=[
            pltpu.VMEM((per_w,), jnp.int32),
            pltpu.VMEM((chunk, d), dtype),
            pltpu.SemaphoreType.DMA,
        ],
    )
    def k(src_hbm, idx_hbm, out_hbm, idx_v, buf, sem):
        wid = lax.axis_index("s") * 2 + lax.axis_index("c")
        base = wid * per_w
        pltpu.sync_copy(idx_hbm.at[pl.ds(base, per_w)], idx_v)

        def body(i, carry):
            off = i * chunk
            pltpu.async_copy(src_hbm.at[idx_v.at[pl.ds(off, chunk)]],
                             buf, sem).wait()
            pltpu.sync_copy(buf, out_hbm.at[pl.ds(base + off, chunk)])
            return carry

        lax.fori_loop(0, n_ch, body, 0)

    return k


def _sc_gather_combine(src, idx):
    return _make_sc_gather(_A, _D // 2, 128, jnp.int32)(src, idx)


def _unpack_halves(p):
    lo = pltpu.unpack_elementwise(p, index=0, packed_dtype=jnp.bfloat16,
                                  unpacked_dtype=jnp.float32)
    hi = pltpu.unpack_elementwise(p, index=1, packed_dtype=jnp.bfloat16,
                                  unpacked_dtype=jnp.float32)
    return jnp.concatenate([lo, hi], axis=1)


def _add_body(a_ref, b_ref, wa_ref, wb_ref, out_ref):
    a = _unpack_halves(a_ref[...])
    b = _unpack_halves(b_ref[...])
    out_ref[...] = a * wa_ref[...] + b * wb_ref[...]


_NBLK = _B // 256


def _combine_add(ys_ab, w1g, w2g):
    return pl.pallas_call(
        _add_body,
        grid=(_NBLK,),
        in_specs=[
            pl.BlockSpec((256, _D // 2), lambda i: (i, 0)),
            pl.BlockSpec((256, _D // 2), lambda i: (i + _NBLK, 0)),
            pl.BlockSpec((256, 1), lambda i: (i, 0)),
            pl.BlockSpec((256, 1), lambda i: (i, 0)),
        ],
        out_specs=pl.BlockSpec((256, _D), lambda i: (i, 0)),
        out_shape=jax.ShapeDtypeStruct((_B, _D), jnp.float32),
    )(ys_ab, ys_ab, w1g, w2g)


def kernel(x, Wg, bg, W1, b1, W2, b2):
    d1c, d2c, w1g, w2g, counts_f, aux = _gate(x, Wg, bg.reshape(1, _E))
    d1 = d1c[:, 0]
    d2 = d2c[:, 0]
    dest = jnp.concatenate([d1, d2], axis=0)

    counts = counts_f[0].astype(jnp.int32)
    ntiles = (counts + _M - 1) // _M
    cum_tiles = jnp.cumsum(ntiles)
    total_tiles = cum_tiles[_E - 1]
    tid = jnp.arange(_TMAX, dtype=jnp.int32)
    tclamp = jnp.minimum(tid, total_tiles - 1)
    texp = jnp.searchsorted(cum_tiles, tclamp, side="right").astype(jnp.int32)
    tvalid = (tid < total_tiles).astype(jnp.int32)
    nz = (ntiles > 0).astype(jnp.int32)
    grpidx = jnp.cumsum(nz) - nz
    eq = (texp[:, None] == jnp.arange(_E, dtype=jnp.int32)[None, :])
    slot_t = jnp.sum(jnp.where(eq, grpidx[None, :], 0), axis=1) % 2
    nf = jnp.sum(jnp.where(eq, cum_tiles[None, :], 0), axis=1)
    has_next = (nf < total_tiles).astype(jnp.int32)
    next_e = jnp.searchsorted(
        cum_tiles, jnp.minimum(nf, total_tiles - 1),
        side="right").astype(jnp.int32)
    prev_exp = jnp.concatenate([jnp.full((1,), -1, jnp.int32), texp[:-1]])
    first_t = (tvalid * (texp != prev_exp)).astype(jnp.int32)
    meta = jnp.stack([texp, tvalid, slot_t, first_t, next_e, has_next],
                     axis=0)

    xs = _make_sc_dispatch()(x, d1, d2)
    ys = _grouped_mlp(meta, xs, W1, b1.reshape(_E, 1, _H),
                      W2, b2.reshape(_E, 1, _D))

    ys_ab = _sc_gather_combine(ys, dest)
    out = _combine_add(ys_ab, w1g, w2g)

    return (out, aux[0, 0])


# speedup vs baseline: 1.7665x; 1.0214x over previous
import functools
import math

import jax
import jax.numpy as jnp
from jax import lax
from jax.experimental import pallas as pl
from jax.experimental.pallas import tpu as pltpu
from jax.experimental.pallas import tpu_sc as plsc

_B, _D, _H, _E, _K = 2048, 1024, 2048, 8, 2
_M = 256
_TMAX = 24
_R = _TMAX * _M
_A = _B * _K
_NW = 32
_EPAD = 128


def _gate_body(x_ref, wg_ref, bg_ref, d1_ref, d2_ref, w1_ref, w2_ref,
               cnt_ref, xp_ref, aux_ref):
    xv = x_ref[...]
    xp_ref[...] = pltpu.pack_elementwise(
        [xv[:, :_D // 2], xv[:, _D // 2:]], packed_dtype=jnp.bfloat16)
    s = jnp.dot(xv, wg_ref[...],
                preferred_element_type=jnp.float32) + bg_ref[...]
    col = lax.broadcasted_iota(jnp.int32, s.shape, 1)
    big = jnp.int32(2 ** 30)
    m1 = jnp.max(s, axis=1, keepdims=True)
    e1 = jnp.min(jnp.where(s >= m1, col, big), axis=1, keepdims=True)
    s2 = jnp.where(col == e1, -jnp.inf, s)
    m2 = jnp.max(s2, axis=1, keepdims=True)
    e2 = jnp.min(jnp.where(s2 >= m2, col, big), axis=1, keepdims=True)
    p = jnp.exp(m2 - m1)
    w1_ref[...] = 1.0 / (1.0 + p)
    w2_ref[...] = p / (1.0 + p)
    oh1 = col == e1
    oh2 = col == e2
    on = jnp.where(oh1, 1.0, 0.0) + jnp.where(oh2, 1.0, 0.0)
    acc = on
    k = 1
    while k < _B:
        acc = acc + jnp.concatenate(
            [jnp.zeros((k, _E), jnp.float32), acc[:-k, :]], axis=0)
        k *= 2
    cum = acc - on
    counts = acc[_B - 1:_B, :]
    ntiles = jnp.floor((counts + (_M - 1)) * (1.0 / _M))
    ct = ntiles
    k = 1
    while k < _E:
        ct = ct + jnp.concatenate(
            [jnp.zeros((1, k), jnp.float32), ct[:, :-k]], axis=1)
        k *= 2
    pad_off = (ct - ntiles) * _M
    po_b = pad_off + cum
    d1_ref[...] = jnp.sum(jnp.where(oh1, po_b, 0.0), axis=1,
                          keepdims=True).astype(jnp.int32)
    d2_ref[...] = jnp.sum(jnp.where(oh2, po_b, 0.0), axis=1,
                          keepdims=True).astype(jnp.int32)
    cnt_ref[...] = counts
    frac = counts * (1.0 / (_B * _K))
    aux_ref[0, 0] = jnp.sum((frac - 1.0 / _E) ** 2) * (1.0 / _E)


def _gate(x, wg, bg_row):
    return pl.pallas_call(
        _gate_body,
        out_shape=(
            jax.ShapeDtypeStruct((_B, 1), jnp.int32),
            jax.ShapeDtypeStruct((_B, 1), jnp.int32),
            jax.ShapeDtypeStruct((_B, 1), jnp.float32),
            jax.ShapeDtypeStruct((_B, 1), jnp.float32),
            jax.ShapeDtypeStruct((1, _E), jnp.float32),
            jax.ShapeDtypeStruct((_B, _D // 2), jnp.int32),
            jax.ShapeDtypeStruct((1, 1), jnp.float32),
        ),
        out_specs=(
            pl.BlockSpec((_B, 1), lambda: (0, 0)),
            pl.BlockSpec((_B, 1), lambda: (0, 0)),
            pl.BlockSpec((_B, 1), lambda: (0, 0)),
            pl.BlockSpec((_B, 1), lambda: (0, 0)),
            pl.BlockSpec((1, _E), lambda: (0, 0)),
            pl.BlockSpec((_B, _D // 2), lambda: (0, 0)),
            pl.BlockSpec(memory_space=pltpu.SMEM),
        ),
    )(x, wg, bg_row)


def _mlp_body(meta_ref, xs_ref, b1_ref, b2_ref, w1_hbm, w2_hbm, out_ref,
              w1b, w2b, s1, s2):
    t = pl.program_id(0)
    e = meta_ref[0, t]
    valid = meta_ref[1, t]
    slot = meta_ref[2, t]
    first = meta_ref[3, t]
    nxt = meta_ref[4, t]
    has_next = meta_ref[5, t]

    @pl.when(t == 0)
    def _():
        pltpu.make_async_copy(w1_hbm.at[e], w1b.at[0], s1.at[0]).start()
        pltpu.make_async_copy(w2_hbm.at[e], w2b.at[0], s2.at[0]).start()

    def _boundary(cs, ns):
        pltpu.make_async_copy(w1_hbm.at[e], w1b.at[cs], s1.at[cs]).wait()
        pltpu.make_async_copy(w2_hbm.at[e], w2b.at[cs], s2.at[cs]).wait()

        @pl.when(has_next == 1)
        def _():
            pltpu.make_async_copy(w1_hbm.at[nxt], w1b.at[ns],
                                  s1.at[ns]).start()
            pltpu.make_async_copy(w2_hbm.at[nxt], w2b.at[ns],
                                  s2.at[ns]).start()

    @pl.when(jnp.logical_and(first == 1, slot == 0))
    def _():
        _boundary(0, 1)

    @pl.when(jnp.logical_and(first == 1, slot == 1))
    def _():
        _boundary(1, 0)

    def _compute(w1v, w2v):
        xg = _unpack_halves(xs_ref[...])
        h = jnp.dot(xg, w1v, preferred_element_type=jnp.float32)
        h = h + b1_ref[0]
        h = 0.5 * h * (1.0 + lax.erf(h * (1.0 / math.sqrt(2.0))))
        y = jnp.dot(h, w2v, preferred_element_type=jnp.float32)
        y = y + b2_ref[0]
        out_ref[...] = pltpu.pack_elementwise(
            [y[:, :_D // 2], y[:, _D // 2:]], packed_dtype=jnp.bfloat16)

    @pl.when(jnp.logical_and(valid == 1, slot == 0))
    def _():
        _compute(w1b[0], w2b[0])

    @pl.when(jnp.logical_and(valid == 1, slot == 1))
    def _():
        _compute(w1b[1], w2b[1])


def _grouped_mlp(meta, xs, w1, b1, w2, b2):
    grid_spec = pltpu.PrefetchScalarGridSpec(
        num_scalar_prefetch=1,
        grid=(_TMAX,),
        in_specs=[
            pl.BlockSpec((_M, _D // 2), lambda t, m: (t, 0)),
            pl.BlockSpec((1, 1, _H), lambda t, m: (m[0, t], 0, 0)),
            pl.BlockSpec((1, 1, _D), lambda t, m: (m[0, t], 0, 0)),
            pl.BlockSpec(memory_space=pl.ANY),
            pl.BlockSpec(memory_space=pl.ANY),
        ],
        out_specs=pl.BlockSpec((_M, _D // 2), lambda t, m: (t, 0)),
        scratch_shapes=[
            pltpu.VMEM((2, _D, _H), jnp.float32),
            pltpu.VMEM((2, _H, _D), jnp.float32),
            pltpu.SemaphoreType.DMA((2,)),
            pltpu.SemaphoreType.DMA((2,)),
        ],
    )
    return pl.pallas_call(
        _mlp_body,
        grid_spec=grid_spec,
        out_shape=jax.ShapeDtypeStruct((_R, _D // 2), jnp.int32),
        compiler_params=pltpu.CompilerParams(
            dimension_semantics=("arbitrary",)),
    )(meta, xs, b1, b2, w1, w2)


@functools.lru_cache(maxsize=None)
def _make_sc_dispatch():
    per_w = _B // _NW
    mesh = plsc.VectorSubcoreMesh(core_axis_name="c", subcore_axis_name="s")

    @functools.partial(
        pl.kernel,
        mesh=mesh,
        out_type=jax.ShapeDtypeStruct((_R, _D // 2), jnp.int32),
        scratch_types=[
            pltpu.VMEM((per_w,), jnp.int32),
            pltpu.VMEM((per_w,), jnp.int32),
            pltpu.VMEM((per_w, _D // 2), jnp.int32),
            pltpu.SemaphoreType.DMA,
            pltpu.SemaphoreType.DMA,
        ],
    )
    def k(x_hbm, d1_hbm, d2_hbm, out_hbm, i1_v, i2_v, buf, s1, s2):
        wid = lax.axis_index("s") * 2 + lax.axis_index("c")
        base = wid * per_w
        pltpu.sync_copy(d1_hbm.at[pl.ds(base, per_w)], i1_v)
        pltpu.sync_copy(d2_hbm.at[pl.ds(base, per_w)], i2_v)
        pltpu.sync_copy(x_hbm.at[pl.ds(base, per_w)], buf)
        c1 = pltpu.async_copy(buf, out_hbm.at[i1_v], s1)
        c2 = pltpu.async_copy(buf, out_hbm.at[i2_v], s2)
        c1.wait()
        c2.wait()

    return k


@functools.lru_cache(maxsize=None)
def _make_sc_gather(n_idx, d, chunk, dtype):
    per_w = n_idx // _NW
    n_ch = per_w // chunk
    mesh = plsc.VectorSubcoreMesh(core_axis_name="c", subcore_axis_name="s")

    @functools.partial(
        pl.kernel,
        mesh=mesh,
        out_type=jax.ShapeDtypeStruct((n_idx, d), dtype),
        scratch_types=[
            pltpu.VMEM((per_w,), jnp.int32),
            pltpu.VMEM((chunk, d), dtype),
            pltpu.SemaphoreType.DMA,
        ],
    )
    def k(src_hbm, idx_hbm, out_hbm, idx_v, buf, sem):
        wid = lax.axis_index("s") * 2 + lax.axis_index("c")
        base = wid * per_w
        pltpu.sync_copy(idx_hbm.at[pl.ds(base, per_w)], idx_v)

        def body(i, carry):
            off = i * chunk
            pltpu.async_copy(src_hbm.at[idx_v.at[pl.ds(off, chunk)]],
                             buf, sem).wait()
            pltpu.sync_copy(buf, out_hbm.at[pl.ds(base + off, chunk)])
            return carry

        lax.fori_loop(0, n_ch, body, 0)

    return k


def _sc_gather_combine(src, idx):
    return _make_sc_gather(_A, _D // 2, 128, jnp.int32)(src, idx)


def _unpack_halves(p):
    lo = pltpu.unpack_elementwise(p, index=0, packed_dtype=jnp.bfloat16,
                                  unpacked_dtype=jnp.float32)
    hi = pltpu.unpack_elementwise(p, index=1, packed_dtype=jnp.bfloat16,
                                  unpacked_dtype=jnp.float32)
    return jnp.concatenate([lo, hi], axis=1)


def _add_body(a_ref, b_ref, wa_ref, wb_ref, out_ref):
    a = _unpack_halves(a_ref[...])
    b = _unpack_halves(b_ref[...])
    out_ref[...] = a * wa_ref[...] + b * wb_ref[...]


_NBLK = _B // 256


def _combine_add(ys_ab, w1g, w2g):
    return pl.pallas_call(
        _add_body,
        grid=(_NBLK,),
        in_specs=[
            pl.BlockSpec((256, _D // 2), lambda i: (i, 0)),
            pl.BlockSpec((256, _D // 2), lambda i: (i + _NBLK, 0)),
            pl.BlockSpec((256, 1), lambda i: (i, 0)),
            pl.BlockSpec((256, 1), lambda i: (i, 0)),
        ],
        out_specs=pl.BlockSpec((256, _D), lambda i: (i, 0)),
        out_shape=jax.ShapeDtypeStruct((_B, _D), jnp.float32),
    )(ys_ab, ys_ab, w1g, w2g)


def kernel(x, Wg, bg, W1, b1, W2, b2):
    d1c, d2c, w1g, w2g, counts_f, xp, aux = _gate(x, Wg, bg.reshape(1, _E))
    d1 = d1c[:, 0]
    d2 = d2c[:, 0]
    dest = jnp.concatenate([d1, d2], axis=0)

    counts = counts_f[0].astype(jnp.int32)
    ntiles = (counts + _M - 1) // _M
    cum_tiles = jnp.cumsum(ntiles)
    total_tiles = cum_tiles[_E - 1]
    tid = jnp.arange(_TMAX, dtype=jnp.int32)
    tclamp = jnp.minimum(tid, total_tiles - 1)
    texp = jnp.searchsorted(cum_tiles, tclamp, side="right").astype(jnp.int32)
    tvalid = (tid < total_tiles).astype(jnp.int32)
    nz = (ntiles > 0).astype(jnp.int32)
    grpidx = jnp.cumsum(nz) - nz
    eq = (texp[:, None] == jnp.arange(_E, dtype=jnp.int32)[None, :])
    slot_t = jnp.sum(jnp.where(eq, grpidx[None, :], 0), axis=1) % 2
    nf = jnp.sum(jnp.where(eq, cum_tiles[None, :], 0), axis=1)
    has_next = (nf < total_tiles).astype(jnp.int32)
    next_e = jnp.searchsorted(
        cum_tiles, jnp.minimum(nf, total_tiles - 1),
        side="right").astype(jnp.int32)
    prev_exp = jnp.concatenate([jnp.full((1,), -1, jnp.int32), texp[:-1]])
    first_t = (tvalid * (texp != prev_exp)).astype(jnp.int32)
    meta = jnp.stack([texp, tvalid, slot_t, first_t, next_e, has_next],
                     axis=0)

    xs = _make_sc_dispatch()(xp, d1, d2)
    ys = _grouped_mlp(meta, xs, W1, b1.reshape(_E, 1, _H),
                      W2, b2.reshape(_E, 1, _D))

    ys_ab = _sc_gather_combine(ys, dest)
    out = _combine_add(ys_ab, w1g, w2g)

    return (out, aux[0, 0])


# speedup vs baseline: 1.7676x; 1.0007x over previous
import functools
import math

import jax
import jax.numpy as jnp
from jax import lax
from jax.experimental import pallas as pl
from jax.experimental.pallas import tpu as pltpu
from jax.experimental.pallas import tpu_sc as plsc

_B, _D, _H, _E, _K = 2048, 1024, 2048, 8, 2
_M = 256
_TMAX = 24
_R = _TMAX * _M
_A = _B * _K
_NW = 32


def _gate_body(x_ref, wg_ref, bg_ref, d1_ref, d2_ref, w1_ref, w2_ref,
               cnt_ref, xp_ref, aux_ref):
    xv = x_ref[...]
    xp_ref[...] = pltpu.pack_elementwise(
        [xv[:, :_D // 2], xv[:, _D // 2:]], packed_dtype=jnp.bfloat16)
    s = jnp.dot(xv, wg_ref[...],
                preferred_element_type=jnp.float32) + bg_ref[...]
    col = lax.broadcasted_iota(jnp.int32, s.shape, 1)
    big = jnp.int32(2 ** 30)
    m1 = jnp.max(s, axis=1, keepdims=True)
    e1 = jnp.min(jnp.where(s >= m1, col, big), axis=1, keepdims=True)
    s2 = jnp.where(col == e1, -jnp.inf, s)
    m2 = jnp.max(s2, axis=1, keepdims=True)
    e2 = jnp.min(jnp.where(s2 >= m2, col, big), axis=1, keepdims=True)
    p = jnp.exp(m2 - m1)
    w1_ref[...] = 1.0 / (1.0 + p)
    w2_ref[...] = p / (1.0 + p)
    oh1 = col == e1
    oh2 = col == e2
    on = jnp.where(oh1, 1.0, 0.0) + jnp.where(oh2, 1.0, 0.0)
    acc = on
    k = 1
    while k < _B:
        acc = acc + jnp.concatenate(
            [jnp.zeros((k, _E), jnp.float32), acc[:-k, :]], axis=0)
        k *= 2
    cum = acc - on
    counts = acc[_B - 1:_B, :]
    ntiles = jnp.floor((counts + (_M - 1)) * (1.0 / _M))
    ct = ntiles
    k = 1
    while k < _E:
        ct = ct + jnp.concatenate(
            [jnp.zeros((1, k), jnp.float32), ct[:, :-k]], axis=1)
        k *= 2
    pad_off = (ct - ntiles) * _M
    po_b = pad_off + cum
    d1_ref[...] = jnp.sum(jnp.where(oh1, po_b, 0.0), axis=1,
                          keepdims=True).astype(jnp.int32)
    d2_ref[...] = jnp.sum(jnp.where(oh2, po_b, 0.0), axis=1,
                          keepdims=True).astype(jnp.int32)
    cnt_ref[...] = counts
    frac = counts * (1.0 / (_B * _K))
    aux_ref[0, 0] = jnp.sum((frac - 1.0 / _E) ** 2) * (1.0 / _E)


def _gate(x, wg, bg_row):
    return pl.pallas_call(
        _gate_body,
        out_shape=(
            jax.ShapeDtypeStruct((_B, 1), jnp.int32),
            jax.ShapeDtypeStruct((_B, 1), jnp.int32),
            jax.ShapeDtypeStruct((_B, 1), jnp.float32),
            jax.ShapeDtypeStruct((_B, 1), jnp.float32),
            jax.ShapeDtypeStruct((1, _E), jnp.float32),
            jax.ShapeDtypeStruct((_B, _D // 2), jnp.int32),
            jax.ShapeDtypeStruct((1, 1), jnp.float32),
        ),
        out_specs=(
            pl.BlockSpec((_B, 1), lambda: (0, 0)),
            pl.BlockSpec((_B, 1), lambda: (0, 0)),
            pl.BlockSpec((_B, 1), lambda: (0, 0)),
            pl.BlockSpec((_B, 1), lambda: (0, 0)),
            pl.BlockSpec((1, _E), lambda: (0, 0)),
            pl.BlockSpec((_B, _D // 2), lambda: (0, 0)),
            pl.BlockSpec(memory_space=pltpu.SMEM),
        ),
    )(x, wg, bg_row)


def _mlp_body(meta_ref, xs_ref, b1_ref, b2_ref, w1_hbm, w2_hbm, out_ref,
              w1b, w2b, s1, s2):
    t = pl.program_id(0)
    e = meta_ref[0, t]
    valid = meta_ref[1, t]
    slot = meta_ref[2, t]
    first = meta_ref[3, t]
    nxt = meta_ref[4, t]
    has_next = meta_ref[5, t]

    @pl.when(t == 0)
    def _():
        pltpu.make_async_copy(w1_hbm.at[e], w1b.at[0], s1.at[0]).start()
        pltpu.make_async_copy(w2_hbm.at[e], w2b.at[0], s2.at[0]).start()

    def _boundary(cs, ns):
        pltpu.make_async_copy(w1_hbm.at[e], w1b.at[cs], s1.at[cs]).wait()
        pltpu.make_async_copy(w2_hbm.at[e], w2b.at[cs], s2.at[cs]).wait()

        @pl.when(has_next == 1)
        def _():
            pltpu.make_async_copy(w1_hbm.at[nxt], w1b.at[ns],
                                  s1.at[ns]).start()
            pltpu.make_async_copy(w2_hbm.at[nxt], w2b.at[ns],
                                  s2.at[ns]).start()

    @pl.when(jnp.logical_and(first == 1, slot == 0))
    def _():
        _boundary(0, 1)

    @pl.when(jnp.logical_and(first == 1, slot == 1))
    def _():
        _boundary(1, 0)

    def _compute(w1v, w2v):
        xg = _unpack_halves(xs_ref[...])
        h = jnp.dot(xg, w1v, preferred_element_type=jnp.float32)
        h = h + b1_ref[0]
        h = 0.5 * h * (1.0 + lax.erf(h * (1.0 / math.sqrt(2.0))))
        y = jnp.dot(h, w2v, preferred_element_type=jnp.float32)
        y = y + b2_ref[0]
        out_ref[...] = pltpu.pack_elementwise(
            [y[:, :_D // 2], y[:, _D // 2:]], packed_dtype=jnp.bfloat16)

    @pl.when(jnp.logical_and(valid == 1, slot == 0))
    def _():
        _compute(w1b[0], w2b[0])

    @pl.when(jnp.logical_and(valid == 1, slot == 1))
    def _():
        _compute(w1b[1], w2b[1])


def _grouped_mlp(meta, xs, w1, b1, w2, b2):
    grid_spec = pltpu.PrefetchScalarGridSpec(
        num_scalar_prefetch=1,
        grid=(_TMAX,),
        in_specs=[
            pl.BlockSpec((_M, _D // 2), lambda t, m: (t, 0)),
            pl.BlockSpec((1, 1, _H), lambda t, m: (m[0, t], 0, 0)),
            pl.BlockSpec((1, 1, _D), lambda t, m: (m[0, t], 0, 0)),
            pl.BlockSpec(memory_space=pl.ANY),
            pl.BlockSpec(memory_space=pl.ANY),
        ],
        out_specs=pl.BlockSpec((_M, _D // 2), lambda t, m: (t, 0)),
        scratch_shapes=[
            pltpu.VMEM((2, _D, _H), jnp.float32),
            pltpu.VMEM((2, _H, _D), jnp.float32),
            pltpu.SemaphoreType.DMA((2,)),
            pltpu.SemaphoreType.DMA((2,)),
        ],
    )
    return pl.pallas_call(
        _mlp_body,
        grid_spec=grid_spec,
        out_shape=jax.ShapeDtypeStruct((_R, _D // 2), jnp.int32),
        compiler_params=pltpu.CompilerParams(
            dimension_semantics=("arbitrary",)),
    )(meta, xs, b1, b2, w1, w2)


@functools.lru_cache(maxsize=None)
def _make_sc_dispatch():
    per_w = _B // _NW
    mesh = plsc.VectorSubcoreMesh(core_axis_name="c", subcore_axis_name="s")

    @functools.partial(
        pl.kernel,
        mesh=mesh,
        out_type=jax.ShapeDtypeStruct((_R, _D // 2), jnp.int32),
        scratch_types=[
            pltpu.VMEM((per_w,), jnp.int32),
            pltpu.VMEM((per_w,), jnp.int32),
            pltpu.VMEM((per_w, _D // 2), jnp.int32),
            pltpu.SemaphoreType.DMA,
            pltpu.SemaphoreType.DMA,
        ],
    )
    def k(x_hbm, d1_hbm, d2_hbm, out_hbm, i1_v, i2_v, buf, s1, s2):
        wid = lax.axis_index("s") * 2 + lax.axis_index("c")
        base = wid * per_w
        pltpu.sync_copy(d1_hbm.at[pl.ds(base, per_w)], i1_v)
        pltpu.sync_copy(d2_hbm.at[pl.ds(base, per_w)], i2_v)
        pltpu.sync_copy(x_hbm.at[pl.ds(base, per_w)], buf)
        c1 = pltpu.async_copy(buf, out_hbm.at[i1_v], s1)
        c2 = pltpu.async_copy(buf, out_hbm.at[i2_v], s2)
        c1.wait()
        c2.wait()

    return k


@functools.lru_cache(maxsize=None)
def _make_sc_gather(n_idx, d, chunk, dtype):
    per_w = n_idx // _NW
    n_ch = per_w // chunk
    mesh = plsc.VectorSubcoreMesh(core_axis_name="c", subcore_axis_name="s")

    @functools.partial(
        pl.kernel,
        mesh=mesh,
        out_type=jax.ShapeDtypeStruct((n_idx, d), dtype),
        scratch_types=[
            pltpu.VMEM((per_w,), jnp.int32),
            pltpu.VMEM((chunk, d), dtype),
            pltpu.SemaphoreType.DMA,
        ],
    )
    def k(src_hbm, idx_hbm, out_hbm, idx_v, buf, sem):
        wid = lax.axis_index("s") * 2 + lax.axis_index("c")
        base = wid * per_w
        pltpu.sync_copy(idx_hbm.at[pl.ds(base, per_w)], idx_v)

        def body(i, carry):
            off = i * chunk
            pltpu.async_copy(src_hbm.at[idx_v.at[pl.ds(off, chunk)]],
                             buf, sem).wait()
            pltpu.sync_copy(buf, out_hbm.at[pl.ds(base + off, chunk)])
            return carry

        lax.fori_loop(0, n_ch, body, 0)

    return k


def _sc_gather_combine(src, idx):
    return _make_sc_gather(_A, _D // 2, 128, jnp.int32)(src, idx)


def _unpack_halves(p):
    lo = pltpu.unpack_elementwise(p, index=0, packed_dtype=jnp.bfloat16,
                                  unpacked_dtype=jnp.float32)
    hi = pltpu.unpack_elementwise(p, index=1, packed_dtype=jnp.bfloat16,
                                  unpacked_dtype=jnp.float32)
    return jnp.concatenate([lo, hi], axis=1)


def _add_body(a_ref, b_ref, wa_ref, wb_ref, out_ref):
    a = _unpack_halves(a_ref[...])
    b = _unpack_halves(b_ref[...])
    out_ref[...] = a * wa_ref[...] + b * wb_ref[...]


_NBLK = _B // 256


def _combine_add(ys_ab, w1g, w2g):
    return pl.pallas_call(
        _add_body,
        grid=(_NBLK,),
        in_specs=[
            pl.BlockSpec((256, _D // 2), lambda i: (i, 0)),
            pl.BlockSpec((256, _D // 2), lambda i: (i + _NBLK, 0)),
            pl.BlockSpec((256, 1), lambda i: (i, 0)),
            pl.BlockSpec((256, 1), lambda i: (i, 0)),
        ],
        out_specs=pl.BlockSpec((256, _D), lambda i: (i, 0)),
        out_shape=jax.ShapeDtypeStruct((_B, _D), jnp.float32),
    )(ys_ab, ys_ab, w1g, w2g)


def kernel(x, Wg, bg, W1, b1, W2, b2):
    d1c, d2c, w1g, w2g, counts_f, xp, aux = _gate(x, Wg, bg.reshape(1, _E))
    d1 = d1c[:, 0]
    d2 = d2c[:, 0]
    dest = jnp.concatenate([d1, d2], axis=0)

    counts = counts_f[0].astype(jnp.int32)
    ntiles = (counts + _M - 1) // _M
    cum_tiles = jnp.cumsum(ntiles)
    total_tiles = cum_tiles[_E - 1]
    tid = jnp.arange(_TMAX, dtype=jnp.int32)
    tclamp = jnp.minimum(tid, total_tiles - 1)
    texp = jnp.searchsorted(cum_tiles, tclamp, side="right").astype(jnp.int32)
    tvalid = (tid < total_tiles).astype(jnp.int32)
    nz = (ntiles > 0).astype(jnp.int32)
    grpidx = jnp.cumsum(nz) - nz
    eq = (texp[:, None] == jnp.arange(_E, dtype=jnp.int32)[None, :])
    slot_t = jnp.sum(jnp.where(eq, grpidx[None, :], 0), axis=1) % 2
    nf = jnp.sum(jnp.where(eq, cum_tiles[None, :], 0), axis=1)
    has_next = (nf < total_tiles).astype(jnp.int32)
    next_e = jnp.searchsorted(
        cum_tiles, jnp.minimum(nf, total_tiles - 1),
        side="right").astype(jnp.int32)
    prev_exp = jnp.concatenate([jnp.full((1,), -1, jnp.int32), texp[:-1]])
    first_t = (tvalid * (texp != prev_exp)).astype(jnp.int32)
    meta = jnp.stack([texp, tvalid, slot_t, first_t, next_e, has_next],
                     axis=0)

    xs = _make_sc_dispatch()(xp, d1, d2)
    ys = _grouped_mlp(meta, xs, W1, b1.reshape(_E, 1, _H),
                      W2, b2.reshape(_E, 1, _D))

    ys_ab = _sc_gather_combine(ys, dest)
    out = _combine_add(ys_ab, w1g, w2g)

    return (out, aux[0, 0])
